# Initial kernel scaffold; baseline (speedup 1.0000x reference)
import jax
import jax.numpy as jnp
from jax.experimental import pallas as pl


def kernel(x, W_conv, codebook):
    raise NotImplementedError("write your pallas kernel here")


# speedup vs baseline: 1.6930x; 1.6930x over previous
import jax
import jax.numpy as jnp
from jax.experimental import pallas as pl


def _vq_body(uv_ref, w0_ref, w1_ref, w2_ref, cb_ref, out_ref):
    C = uv_ref.shape[2]
    u = uv_ref[0, 0]
    v = uv_ref[0, 1]
    v_prev = jnp.concatenate(
        [jnp.zeros((C, 1), jnp.bfloat16), v[:, :-1]], axis=1)
    y = (
        jnp.dot(w1_ref[...], u, preferred_element_type=jnp.float32)
        + jnp.dot(w2_ref[...], v, preferred_element_type=jnp.float32)
        + jnp.dot(w0_ref[...], v_prev, preferred_element_type=jnp.float32)
    )
    yn2 = jnp.sum(y * y, axis=0, keepdims=True)
    enc = (y * (1.0 / jnp.maximum(jnp.sqrt(yn2), 1e-12))).astype(jnp.bfloat16)
    cb = cb_ref[...]
    ss = jnp.sum(cb * cb, axis=1, keepdims=True)
    cbn = cb * (1.0 / jnp.maximum(jnp.sqrt(ss), 1e-12))
    half_bias = 0.5 * jnp.sum(cbn * cbn, axis=1, keepdims=True)
    scores = jnp.dot(cbn.astype(jnp.bfloat16), enc,
                     preferred_element_type=jnp.float32)
    idx = jnp.argmax(scores - half_bias, axis=0)
    out_ref[0] = idx.reshape(1, -1)


@jax.jit
def kernel(x, W_conv, codebook):
    B, C, L = x.shape
    O = W_conv.shape[0]
    K, D = codebook.shape
    T = L // 2
    W0 = W_conv[:, :, 0].astype(jnp.bfloat16)
    W1 = W_conv[:, :, 1].astype(jnp.bfloat16)
    W2 = W_conv[:, :, 2].astype(jnp.bfloat16)
    uv = jnp.moveaxis(x.reshape(B, C, T, 2), 3, 1).astype(jnp.bfloat16)
    out = pl.pallas_call(
        _vq_body,
        grid=(B,),
        in_specs=[
            pl.BlockSpec((1, 2, C, T), lambda b: (b, 0, 0, 0)),
            pl.BlockSpec((O, C), lambda b: (0, 0)),
            pl.BlockSpec((O, C), lambda b: (0, 0)),
            pl.BlockSpec((O, C), lambda b: (0, 0)),
            pl.BlockSpec((K, D), lambda b: (0, 0)),
        ],
        out_specs=pl.BlockSpec((1, 1, T), lambda b: (b, 0, 0)),
        out_shape=jax.ShapeDtypeStruct((B, 1, T), jnp.int32),
    )(uv, W0, W1, W2, codebook)
    return out.reshape(B, T)


# speedup vs baseline: 1.7223x; 1.0173x over previous
import jax
import jax.numpy as jnp
from jax.experimental import pallas as pl
from jax.experimental.pallas import tpu as pltpu


def _vq_body(uv_ref, w0_ref, w1_ref, w2_ref, cb_ref, out_ref):
    C = uv_ref.shape[2]
    u = uv_ref[0, 0]
    v = uv_ref[0, 1]
    v_prev = jnp.concatenate(
        [jnp.zeros((C, 1), jnp.bfloat16), v[:, :-1]], axis=1)
    y = (
        jnp.dot(w1_ref[...], u, preferred_element_type=jnp.float32)
        + jnp.dot(w2_ref[...], v, preferred_element_type=jnp.float32)
        + jnp.dot(w0_ref[...], v_prev, preferred_element_type=jnp.float32)
    )
    yn2 = jnp.sum(y * y, axis=0, keepdims=True)
    enc = (y * (1.0 / jnp.maximum(jnp.sqrt(yn2), 1e-12))).astype(jnp.bfloat16)
    cb = cb_ref[...]
    ss = jnp.sum(cb * cb, axis=1, keepdims=True)
    cbn = cb * (1.0 / jnp.maximum(jnp.sqrt(ss), 1e-12))
    scores = jnp.dot(cbn.astype(jnp.bfloat16), enc,
                     preferred_element_type=jnp.float32)
    idx = jnp.argmax(scores, axis=0)
    out_ref[0] = idx.reshape(1, -1)


@jax.jit
def kernel(x, W_conv, codebook):
    B, C, L = x.shape
    O = W_conv.shape[0]
    K, D = codebook.shape
    T = L // 2
    W0 = W_conv[:, :, 0].astype(jnp.bfloat16)
    W1 = W_conv[:, :, 1].astype(jnp.bfloat16)
    W2 = W_conv[:, :, 2].astype(jnp.bfloat16)
    uv = jnp.moveaxis(x.reshape(B, C, T, 2), 3, 1).astype(jnp.bfloat16)
    out = pl.pallas_call(
        _vq_body,
        grid=(B,),
        in_specs=[
            pl.BlockSpec((1, 2, C, T), lambda b: (b, 0, 0, 0)),
            pl.BlockSpec((O, C), lambda b: (0, 0)),
            pl.BlockSpec((O, C), lambda b: (0, 0)),
            pl.BlockSpec((O, C), lambda b: (0, 0)),
            pl.BlockSpec((K, D), lambda b: (0, 0)),
        ],
        out_specs=pl.BlockSpec((1, 1, T), lambda b: (b, 0, 0)),
        out_shape=jax.ShapeDtypeStruct((B, 1, T), jnp.int32),
        compiler_params=pltpu.CompilerParams(
            dimension_semantics=("parallel",),
        ),
    )(uv, W0, W1, W2, codebook)
    return out.reshape(B, T)


# speedup vs baseline: 1.7451x; 1.0133x over previous
import jax
import jax.numpy as jnp
from jax.experimental import pallas as pl
from jax.experimental.pallas import tpu as pltpu


def _vq_body(uv_ref, w0_ref, w1_ref, w2_ref, cb_ref, out_ref):
    C = uv_ref.shape[2]
    u = uv_ref[0, 0]
    v = uv_ref[0, 1]
    v_prev = jnp.concatenate(
        [jnp.zeros((C, 1), jnp.bfloat16), v[:, :-1]], axis=1)
    y = (
        jnp.dot(w1_ref[...], u, preferred_element_type=jnp.float32)
        + jnp.dot(w2_ref[...], v, preferred_element_type=jnp.float32)
        + jnp.dot(w0_ref[...], v_prev, preferred_element_type=jnp.float32)
    )
    yn2 = jnp.sum(y * y, axis=0, keepdims=True)
    enc = (y * (1.0 / jnp.maximum(jnp.sqrt(yn2), 1e-12))).astype(jnp.bfloat16)
    cb = cb_ref[...]
    ss = jnp.sum(cb * cb, axis=1, keepdims=True)
    cbn = cb * (1.0 / jnp.maximum(jnp.sqrt(ss), 1e-12))
    scores = jnp.dot(cbn.astype(jnp.bfloat16), enc,
                     preferred_element_type=jnp.float32)
    idx = jnp.argmax(scores, axis=0)
    out_ref[0] = idx.reshape(1, -1)


@jax.jit
def kernel(x, W_conv, codebook):
    B, C, L = x.shape
    O = W_conv.shape[0]
    K, D = codebook.shape
    T = L // 2
    W0 = W_conv[:, :, 0].astype(jnp.bfloat16)
    W1 = W_conv[:, :, 1].astype(jnp.bfloat16)
    W2 = W_conv[:, :, 2].astype(jnp.bfloat16)
    uv = jnp.moveaxis(x.reshape(B, C, T, 2), 3, 1).astype(jnp.bfloat16)
    out = pl.pallas_call(
        _vq_body,
        grid=(B,),
        in_specs=[
            pl.BlockSpec((1, 2, C, T), lambda b: (b, 0, 0, 0)),
            pl.BlockSpec((O, C), lambda b: (0, 0)),
            pl.BlockSpec((O, C), lambda b: (0, 0)),
            pl.BlockSpec((O, C), lambda b: (0, 0)),
            pl.BlockSpec((K, D), lambda b: (0, 0)),
        ],
        out_specs=pl.BlockSpec((1, 1, T), lambda b: (b, 0, 0)),
        out_shape=jax.ShapeDtypeStruct((B, 1, T), jnp.int32),
        compiler_params=pltpu.CompilerParams(
            dimension_semantics=("parallel",),
            allow_input_fusion=[True, False, False, False, False],
        ),
    )(uv, W0, W1, W2, codebook)
    return out.reshape(B, T)


# speedup vs baseline: 2.9105x; 1.6678x over previous
import jax
import jax.numpy as jnp
from jax import lax
from jax.experimental import pallas as pl
from jax.experimental.pallas import tpu as pltpu


def _vq_body(x_ref, w0_ref, w1_ref, w2_ref, cb_ref, out_ref):
    xb = x_ref[0]
    C, L = xb.shape
    xc = xb.astype(jnp.bfloat16)
    xn = pltpu.roll(xc, L - 1, 1)
    xp = pltpu.roll(xc, 1, 1)
    lane = lax.broadcasted_iota(jnp.int32, (C, L), 1)
    xp = jnp.where(lane == 0, jnp.bfloat16(0), xp)
    y = (
        jnp.dot(w1_ref[...], xc, preferred_element_type=jnp.float32)
        + jnp.dot(w2_ref[...], xn, preferred_element_type=jnp.float32)
        + jnp.dot(w0_ref[...], xp, preferred_element_type=jnp.float32)
    )
    n2 = jnp.sum(y * y, axis=0, keepdims=True)
    enc = (y * (1.0 / jnp.maximum(jnp.sqrt(n2), 1e-12))).astype(jnp.bfloat16)
    cb = cb_ref[...]
    ss = jnp.sum(cb * cb, axis=1, keepdims=True)
    cbn = (cb * (1.0 / jnp.maximum(jnp.sqrt(ss), 1e-12))).astype(jnp.bfloat16)
    scores = jnp.dot(cbn, enc, preferred_element_type=jnp.float32)
    idx = jnp.argmax(scores, axis=0)
    out_ref[0] = idx.reshape(1, -1)


@jax.jit
def kernel(x, W_conv, codebook):
    B, C, L = x.shape
    O = W_conv.shape[0]
    K, D = codebook.shape
    W0 = W_conv[:, :, 0].astype(jnp.bfloat16)
    W1 = W_conv[:, :, 1].astype(jnp.bfloat16)
    W2 = W_conv[:, :, 2].astype(jnp.bfloat16)
    out = pl.pallas_call(
        _vq_body,
        grid=(B,),
        in_specs=[
            pl.BlockSpec((1, C, L), lambda b: (b, 0, 0)),
            pl.BlockSpec((O, C), lambda b: (0, 0)),
            pl.BlockSpec((O, C), lambda b: (0, 0)),
            pl.BlockSpec((O, C), lambda b: (0, 0)),
            pl.BlockSpec((K, D), lambda b: (0, 0)),
        ],
        out_specs=pl.BlockSpec((1, 1, L), lambda b: (b, 0, 0)),
        out_shape=jax.ShapeDtypeStruct((B, 1, L), jnp.int32),
        compiler_params=pltpu.CompilerParams(
            dimension_semantics=("parallel",),
        ),
    )(x, W0, W1, W2, codebook)
    return out[:, 0, 0::2]


# speedup vs baseline: 3.9195x; 1.3467x over previous
import jax
import jax.numpy as jnp
from jax import lax
from jax.experimental import pallas as pl
from jax.experimental.pallas import tpu as pltpu


def _vq_body(x_ref, w0_ref, w1_ref, w2_ref, cb_ref, out_ref, encT_ref):
    xb = x_ref[0]
    C, L = xb.shape
    xc = xb.astype(jnp.bfloat16)
    xn = pltpu.roll(xc, L - 1, 1)
    xp = pltpu.roll(xc, 1, 1)
    lane = lax.broadcasted_iota(jnp.int32, (C, L), 1)
    xp = jnp.where(lane == 0, jnp.bfloat16(0), xp)
    y = (
        jnp.dot(w1_ref[...], xc, preferred_element_type=jnp.float32)
        + jnp.dot(w2_ref[...], xn, preferred_element_type=jnp.float32)
        + jnp.dot(w0_ref[...], xp, preferred_element_type=jnp.float32)
    )
    n2 = jnp.sum(y * y, axis=0, keepdims=True)
    enc = y * (1.0 / jnp.maximum(jnp.sqrt(n2), 1e-12))
    encT_ref[...] = enc.T
    encT_even = encT_ref[pl.Slice(0, L // 2, 2), :].astype(jnp.bfloat16)
    cb = cb_ref[...]
    ss = jnp.sum(cb * cb, axis=1, keepdims=True)
    cbn = (cb * (1.0 / jnp.maximum(jnp.sqrt(ss), 1e-12))).astype(jnp.bfloat16)
    scores = lax.dot_general(
        cbn,
        encT_even,
        (((1,), (1,)), ((), ())),
        preferred_element_type=jnp.float32,
    )
    idx = jnp.argmax(scores, axis=0)
    out_ref[0] = idx.reshape(1, -1)


@jax.jit
def kernel(x, W_conv, codebook):
    B, C, L = x.shape
    O = W_conv.shape[0]
    K, D = codebook.shape
    W0 = W_conv[:, :, 0].astype(jnp.bfloat16)
    W1 = W_conv[:, :, 1].astype(jnp.bfloat16)
    W2 = W_conv[:, :, 2].astype(jnp.bfloat16)
    out = pl.pallas_call(
        _vq_body,
        grid=(B,),
        in_specs=[
            pl.BlockSpec((1, C, L), lambda b: (b, 0, 0)),
            pl.BlockSpec((O, C), lambda b: (0, 0)),
            pl.BlockSpec((O, C), lambda b: (0, 0)),
            pl.BlockSpec((O, C), lambda b: (0, 0)),
            pl.BlockSpec((K, D), lambda b: (0, 0)),
        ],
        out_specs=pl.BlockSpec((1, 1, L // 2), lambda b: (b, 0, 0)),
        out_shape=jax.ShapeDtypeStruct((B, 1, L // 2), jnp.int32),
        scratch_shapes=[pltpu.VMEM((L, O), jnp.float32)],
        compiler_params=pltpu.CompilerParams(
            dimension_semantics=("parallel",),
        ),
    )(x, W0, W1, W2, codebook)
    return out.reshape(B, L // 2)


# speedup vs baseline: 4.0217x; 1.0261x over previous
import jax
import jax.numpy as jnp
from jax import lax
from jax.experimental import pallas as pl
from jax.experimental.pallas import tpu as pltpu


def _vq_body(x_ref, w0_ref, w1_ref, w2_ref, cb_ref, out_ref, encT_ref):
    xb = x_ref[0]
    C, L = xb.shape
    xc = xb.astype(jnp.bfloat16)
    s1 = jnp.dot(w1_ref[...], xc, preferred_element_type=jnp.float32)
    s2 = jnp.dot(w2_ref[...], xc, preferred_element_type=jnp.float32)
    s0 = jnp.dot(w0_ref[...], xc, preferred_element_type=jnp.float32)
    O = s1.shape[0]
    s0r = pltpu.roll(s0, 1, 1)
    lane = lax.broadcasted_iota(jnp.int32, (O, L), 1)
    s0r = jnp.where(lane == 0, jnp.float32(0), s0r)
    y = s1 + pltpu.roll(s2, L - 1, 1) + s0r
    n2 = jnp.sum(y * y, axis=0, keepdims=True)
    enc = y * (1.0 / jnp.maximum(jnp.sqrt(n2), 1e-12))
    encT_ref[...] = enc.T
    encT_even = encT_ref[pl.Slice(0, L // 2, 2), :].astype(jnp.bfloat16)
    cb = cb_ref[...]
    ss = jnp.sum(cb * cb, axis=1, keepdims=True)
    cbn = (cb * (1.0 / jnp.maximum(jnp.sqrt(ss), 1e-12))).astype(jnp.bfloat16)
    scores = lax.dot_general(
        cbn,
        encT_even,
        (((1,), (1,)), ((), ())),
        preferred_element_type=jnp.float32,
    )
    idx = jnp.argmax(scores, axis=0)
    out_ref[0] = idx.reshape(1, -1)


@jax.jit
def kernel(x, W_conv, codebook):
    B, C, L = x.shape
    O = W_conv.shape[0]
    K, D = codebook.shape
    W0 = W_conv[:, :, 0].astype(jnp.bfloat16)
    W1 = W_conv[:, :, 1].astype(jnp.bfloat16)
    W2 = W_conv[:, :, 2].astype(jnp.bfloat16)
    out = pl.pallas_call(
        _vq_body,
        grid=(B,),
        in_specs=[
            pl.BlockSpec((1, C, L), lambda b: (b, 0, 0)),
            pl.BlockSpec((O, C), lambda b: (0, 0)),
            pl.BlockSpec((O, C), lambda b: (0, 0)),
            pl.BlockSpec((O, C), lambda b: (0, 0)),
            pl.BlockSpec((K, D), lambda b: (0, 0)),
        ],
        out_specs=pl.BlockSpec((1, 1, L // 2), lambda b: (b, 0, 0)),
        out_shape=jax.ShapeDtypeStruct((B, 1, L // 2), jnp.int32),
        scratch_shapes=[pltpu.VMEM((L, O), jnp.float32)],
        compiler_params=pltpu.CompilerParams(
            dimension_semantics=("parallel",),
        ),
    )(x, W0, W1, W2, codebook)
    return out.reshape(B, L // 2)
